# baseline (device time: 20320 ns/iter reference)
import jax
import jax.numpy as jnp
from jax import lax
from jax.experimental import pallas as pl
from jax.experimental.pallas import tpu as pltpu

B, HS, WS, C = 2, 64, 64, 64
GN = 128 * 128


def kernel(x, k, Wp):
    def body(x_ref, k_ref, wp_ref, out_ref,
             p_ref, stats_ref, rs_send, rs_recv, cs_send, cs_recv,
             send_sems, recv_sems):
        my_x = lax.axis_index("x")
        my_y = lax.axis_index("y")
        x_nbr = (1 - my_x, my_y)
        y_nbr = (my_x, 1 - my_y)

        barrier = pltpu.get_barrier_semaphore()
        pl.semaphore_signal(barrier, inc=1, device_id=x_nbr,
                            device_id_type=pl.DeviceIdType.MESH)
        pl.semaphore_signal(barrier, inc=1, device_id=y_nbr,
                            device_id_type=pl.DeviceIdType.MESH)
        pl.semaphore_wait(barrier, 2)

        xv = x_ref[...]

        s = jnp.sum(xv, axis=(1, 2))
        ss = jnp.sum(xv * xv, axis=(1, 2))
        stats_ref[0, :, :] = jnp.concatenate([s, ss], axis=0)

        stats_x = pltpu.make_async_remote_copy(
            src_ref=stats_ref.at[0], dst_ref=stats_ref.at[1],
            send_sem=send_sems.at[0], recv_sem=recv_sems.at[0],
            device_id=x_nbr, device_id_type=pl.DeviceIdType.MESH)
        stats_x.start()

        @pl.when(my_x == 0)
        def _():
            rs_send[...] = x_ref[:, HS - 1:HS, :, :]

        @pl.when(my_x == 1)
        def _():
            rs_send[...] = x_ref[:, 0:1, :, :]

        row = pltpu.make_async_remote_copy(
            src_ref=rs_send, dst_ref=rs_recv,
            send_sem=send_sems.at[2], recv_sem=recv_sems.at[2],
            device_id=x_nbr, device_id_type=pl.DeviceIdType.MESH)
        row.start()

        p_ref[:, 1:HS + 1, 1:WS + 1, :] = xv

        @pl.when(my_x == 0)
        def _():
            p_ref[:, 0:1, 1:WS + 1, :] = xv[:, 0:1, :, :]

        @pl.when(my_x == 1)
        def _():
            p_ref[:, HS + 1:HS + 2, 1:WS + 1, :] = xv[:, HS - 1:HS, :, :]

        stats_x.wait()
        stats_ref[2, :, :] = stats_ref[0, :, :] + stats_ref[1, :, :]
        stats_y = pltpu.make_async_remote_copy(
            src_ref=stats_ref.at[2], dst_ref=stats_ref.at[3],
            send_sem=send_sems.at[1], recv_sem=recv_sems.at[1],
            device_id=y_nbr, device_id_type=pl.DeviceIdType.MESH)
        stats_y.start()

        row.wait()

        @pl.when(my_x == 0)
        def _():
            p_ref[:, HS + 1:HS + 2, 1:WS + 1, :] = rs_recv[...]

        @pl.when(my_x == 1)
        def _():
            p_ref[:, 0:1, 1:WS + 1, :] = rs_recv[...]

        @pl.when(my_y == 0)
        def _():
            cs_send[...] = p_ref[:, :, WS:WS + 1, :]
            p_ref[:, :, 0:1, :] = p_ref[:, :, 1:2, :]

        @pl.when(my_y == 1)
        def _():
            cs_send[...] = p_ref[:, :, 1:2, :]
            p_ref[:, :, WS + 1:WS + 2, :] = p_ref[:, :, WS:WS + 1, :]

        col = pltpu.make_async_remote_copy(
            src_ref=cs_send, dst_ref=cs_recv,
            send_sem=send_sems.at[3], recv_sem=recv_sems.at[3],
            device_id=y_nbr, device_id_type=pl.DeviceIdType.MESH)
        col.start()

        stats_y.wait()
        tot = stats_ref[2, :, :] + stats_ref[3, :, :]
        mean = tot[0:B] * (1.0 / GN)
        var = tot[B:2 * B] * (1.0 / GN) - mean * mean
        inv = lax.rsqrt(var + 1e-5)

        col.wait()

        @pl.when(my_y == 0)
        def _():
            p_ref[:, :, WS + 1:WS + 2, :] = cs_recv[...]

        @pl.when(my_y == 1)
        def _():
            p_ref[:, :, 0:1, :] = cs_recv[...]

        pv = p_ref[...]
        h = (pv - mean[:, None, None, :]) * inv[:, None, None, :]
        kv = k_ref[...]
        conv = jnp.zeros((B, HS, WS, C), jnp.float32)
        for di in range(3):
            for dj in range(3):
                conv = conv + (h[:, di:di + HS, dj:dj + WS, :]
                               * kv[di, dj][None, None, None, :])
        a = conv * (1.0 / (1.0 + jnp.exp(-conv)))
        proj = jnp.dot(a.reshape(B * HS * WS, C), wp_ref[...],
                       preferred_element_type=jnp.float32)
        out_ref[...] = xv + proj.reshape(B, HS, WS, C)

    return pl.pallas_call(
        body,
        out_shape=jax.ShapeDtypeStruct((B, HS, WS, C), jnp.float32),
        in_specs=[
            pl.BlockSpec(memory_space=pltpu.VMEM),
            pl.BlockSpec(memory_space=pltpu.VMEM),
            pl.BlockSpec(memory_space=pltpu.VMEM),
        ],
        out_specs=pl.BlockSpec(memory_space=pltpu.VMEM),
        scratch_shapes=[
            pltpu.VMEM((B, HS + 2, WS + 2, C), jnp.float32),
            pltpu.VMEM((4, 2 * B, C), jnp.float32),
            pltpu.VMEM((B, 1, WS, C), jnp.float32),
            pltpu.VMEM((B, 1, WS, C), jnp.float32),
            pltpu.VMEM((B, HS + 2, 1, C), jnp.float32),
            pltpu.VMEM((B, HS + 2, 1, C), jnp.float32),
            pltpu.SemaphoreType.DMA((4,)),
            pltpu.SemaphoreType.DMA((4,)),
        ],
        compiler_params=pltpu.CompilerParams(collective_id=0),
    )(x, k, Wp)
